# baseline (speedup 1.0000x reference)
import jax
import jax.numpy as jnp
from jax import lax
from jax.experimental import pallas as pl
from jax.experimental.pallas import tpu as pltpu
from jax.experimental.pallas import tpu_sc as plsc

N = 100000
DIM = 32
NW = 32
B_PER_W = 3128
N_PAD = NW * B_PER_W


def _emb_body(table_hbm, idx_hbm, out_hbm, idx_v, rows_v, sem):
    wid = lax.axis_index("s") * 2 + lax.axis_index("c")
    base = wid * B_PER_W
    pltpu.sync_copy(idx_hbm.at[pl.ds(base, B_PER_W)], idx_v)
    pltpu.async_copy(table_hbm.at[idx_v], rows_v, sem).wait()
    pltpu.sync_copy(rows_v, out_hbm.at[pl.ds(base, B_PER_W)])


def _embed(table, idx_pad):
    mesh = plsc.VectorSubcoreMesh(core_axis_name="c", subcore_axis_name="s")
    f = pl.kernel(
        _emb_body,
        out_type=jax.ShapeDtypeStruct((N_PAD, DIM), jnp.float32),
        mesh=mesh,
        scratch_types=[
            pltpu.VMEM((B_PER_W,), jnp.int32),
            pltpu.VMEM((B_PER_W, DIM), jnp.float32),
            pltpu.SemaphoreType.DMA,
        ],
        compiler_params=pltpu.CompilerParams(use_tc_tiling_on_sc=False),
    )
    return f(table, idx_pad)


def kernel(x, edge_index, edge_attr, batch, depth, ptr, table):
    idx = x.reshape(-1)
    idx_pad = jnp.pad(idx, (0, N_PAD - N))
    out = _embed(table, idx_pad)
    return (out[:N], edge_index, edge_attr, batch, depth, ptr)


# speedup vs baseline: 1.1336x; 1.1336x over previous
import jax
import jax.numpy as jnp
from jax import lax
from jax.experimental import pallas as pl
from jax.experimental.pallas import tpu as pltpu
from jax.experimental.pallas import tpu_sc as plsc

N = 100000
DIM = 32
ROWS_PAD = 1001472
NW = 32
B_PER_W = 3128
N_PAD = NW * B_PER_W

TCOLS = 2048
QUART = TCOLS // 4
OUT_BLK = TCOLS * DIM // 128
SCRATCH_ROWS = ROWS_PAD * DIM // 128
GRID = -(-ROWS_PAD // TCOLS)


def _transpose_body(x_ref, y_ref):
    z = jnp.transpose(x_ref[...], (1, 0))
    y_ref[...] = jnp.concatenate(
        [z[0:QUART], z[QUART:2 * QUART], z[2 * QUART:3 * QUART],
         z[3 * QUART:4 * QUART]], axis=1)


def _detile(table_t):
    return pl.pallas_call(
        _transpose_body,
        grid=(GRID,),
        in_specs=[pl.BlockSpec((DIM, TCOLS), lambda t: (0, t))],
        out_specs=pl.BlockSpec((OUT_BLK, 128), lambda t: (t, 0)),
        out_shape=jax.ShapeDtypeStruct((SCRATCH_ROWS, 128), jnp.float32),
    )(table_t)


def _gather_body(table_hbm, idx_hbm, out_hbm, idx_v, rows_v, sem):
    wid = lax.axis_index("s") * 2 + lax.axis_index("c")
    base = wid * B_PER_W
    pltpu.sync_copy(idx_hbm.at[pl.ds(base, B_PER_W)], idx_v)
    pltpu.async_copy(table_hbm.at[idx_v], rows_v, sem).wait()
    pltpu.sync_copy(rows_v, out_hbm.at[pl.ds(base, B_PER_W)])


def _gather(table_rows, idx_pad):
    mesh = plsc.VectorSubcoreMesh(core_axis_name="c", subcore_axis_name="s")
    f = pl.kernel(
        _gather_body,
        out_type=jax.ShapeDtypeStruct((N_PAD, DIM), jnp.float32),
        mesh=mesh,
        scratch_types=[
            pltpu.VMEM((B_PER_W,), jnp.int32),
            pltpu.VMEM((B_PER_W, DIM), jnp.float32),
            pltpu.SemaphoreType.DMA,
        ],
        compiler_params=pltpu.CompilerParams(use_tc_tiling_on_sc=False),
    )
    return f(table_rows, idx_pad)


def kernel(x, edge_index, edge_attr, batch, depth, ptr, table):
    table_t = table.T
    scratch = _detile(table_t)
    table_rows = scratch.reshape(ROWS_PAD, DIM)
    idx = x.reshape(-1)
    u = idx & (TCOLS - 1)
    gidx = (idx - u) + ((u & (QUART - 1)) << 2) + (u >> 9)
    gidx_pad = jnp.pad(gidx, (0, N_PAD - N))
    out = _gather(table_rows, gidx_pad)
    return (out[:N], edge_index, edge_attr, batch, depth, ptr)


# speedup vs baseline: 2.0888x; 1.8427x over previous
import jax
import jax.numpy as jnp
from jax import lax
from jax.experimental import pallas as pl
from jax.experimental.pallas import tpu as pltpu
from jax.experimental.pallas import tpu_sc as plsc

N = 100000
DIM = 32
ROWS_PAD = 1007616
NW = 32
B_PER_W = 3128
N_PAD = NW * B_PER_W

TCOLS = 8192
QUART = TCOLS // 4
QUART_LOG2 = QUART.bit_length() - 1
OUT_BLK = TCOLS * DIM // 128
SCRATCH_ROWS = ROWS_PAD * DIM // 128
GRID = -(-ROWS_PAD // TCOLS)


def _transpose_body(x_ref, y_ref):
    x = x_ref[...]
    x4 = jnp.concatenate(
        [x[:, 0:QUART], x[:, QUART:2 * QUART], x[:, 2 * QUART:3 * QUART],
         x[:, 3 * QUART:4 * QUART]], axis=0)
    y_ref[...] = jnp.transpose(x4, (1, 0))


def _detile(table_t):
    return pl.pallas_call(
        _transpose_body,
        grid=(GRID,),
        in_specs=[pl.BlockSpec((DIM, TCOLS), lambda t: (0, t))],
        out_specs=pl.BlockSpec((OUT_BLK, 128), lambda t: (t, 0)),
        out_shape=jax.ShapeDtypeStruct((SCRATCH_ROWS, 128), jnp.float32),
    )(table_t)


def _gather_body(table_hbm, idx_hbm, out_hbm, idx_v, rows_v, sem):
    wid = lax.axis_index("s") * 2 + lax.axis_index("c")
    base = wid * B_PER_W
    pltpu.sync_copy(idx_hbm.at[pl.ds(base, B_PER_W)], idx_v)
    pltpu.async_copy(table_hbm.at[idx_v], rows_v, sem).wait()
    pltpu.sync_copy(rows_v, out_hbm.at[pl.ds(base, B_PER_W)])


def _gather(table_rows, idx_pad):
    mesh = plsc.VectorSubcoreMesh(core_axis_name="c", subcore_axis_name="s")
    f = pl.kernel(
        _gather_body,
        out_type=jax.ShapeDtypeStruct((N_PAD, DIM), jnp.float32),
        mesh=mesh,
        scratch_types=[
            pltpu.VMEM((B_PER_W,), jnp.int32),
            pltpu.VMEM((B_PER_W, DIM), jnp.float32),
            pltpu.SemaphoreType.DMA,
        ],
        compiler_params=pltpu.CompilerParams(use_tc_tiling_on_sc=False),
    )
    return f(table_rows, idx_pad)


def kernel(x, edge_index, edge_attr, batch, depth, ptr, table):
    table_t = table.T
    scratch = _detile(table_t)
    table_rows = scratch.reshape(ROWS_PAD, DIM)
    idx = x.reshape(-1)
    u = idx & (TCOLS - 1)
    gidx = (idx - u) + ((u & (QUART - 1)) << 2) + (u >> QUART_LOG2)
    gidx_pad = jnp.pad(gidx, (0, N_PAD - N))
    out = _gather(table_rows, gidx_pad)
    return (out[:N], edge_index, edge_attr, batch, depth, ptr)
